# baseline (speedup 1.0000x reference)
import functools

import jax
import jax.numpy as jnp
from jax import lax
from jax.experimental import pallas as pl
from jax.experimental.pallas import tpu as pltpu
from jax.experimental.pallas import tpu_sc as plsc

_NC = 2
_NS = 16
_NW = _NC * _NS
_L = 16

_NI = 16384
_NJ = 200
_D = 4
_JB = 8
_IW = _NI // _NW
_KI = _JB * _IW
_KO = _KI * _D
_OROW = _IW * _D
_UNITS = _NJ // _JB


def _make_sc_gather():
    mesh = plsc.VectorSubcoreMesh(
        core_axis_name="c", subcore_axis_name="s", num_cores=_NC,
        num_subcores=_NS)

    @functools.partial(
        pl.kernel,
        out_type=jax.ShapeDtypeStruct((_NI * _NJ * _D,), jnp.float32),
        mesh=mesh,
        compiler_params=pltpu.CompilerParams(needs_layout_passes=False),
        scratch_types=(
            [pltpu.VMEM((_JB, _IW), jnp.int32) for _ in range(4)]
            + [pltpu.VMEM((_KO,), jnp.float32) for _ in range(3)]
            + [
                pltpu.VMEM((32,), jnp.float32),
                pltpu.SemaphoreType.DMA,
                pltpu.SemaphoreType.DMA,
            ]
        ),
    )
    def sc_gather(xt_hbm, wc_hbm, out_hbm, idx0_v, idx1_v, idx2_v, idx3_v,
                  out0_v, out1_v, out2_v, w_v, in_sem, out_sem):
        idx_bufs = [idx0_v, idx1_v, idx2_v, idx3_v]
        out_bufs = [out0_v, out1_v, out2_v]
        wid = lax.axis_index("s") * _NC + lax.axis_index("c")
        i0 = wid * _IW
        pltpu.sync_copy(wc_hbm, w_v)

        def in_copy(n, buf):
            return pltpu.async_copy(
                xt_hbm.at[pl.ds(n * _JB, _JB), pl.ds(i0, _IW)],
                idx_bufs[buf], in_sem)

        def out_copies_for(n, buf):
            return [
                pltpu.async_copy(
                    out_bufs[buf].at[pl.ds(jj * _OROW, _OROW)],
                    out_hbm.at[pl.ds(
                        (n * _JB + jj) * (_NI * _D) + wid * _OROW, _OROW)],
                    out_sem)
                for jj in range(_JB)]

        def compute(ibuf, obuf):
            def inner(m, _):
                jj = m // (_IW // 128)
                tt = m % (_IW // 128)
                ibase = tt * 128
                obase = jj * _OROW + tt * 512
                for c in range(128 // _L):
                    idx = idx_bufs[ibuf][jj, pl.ds(ibase + c * _L, _L)]
                    for dd in range(_D):
                        vals = plsc.load_gather(w_v, [idx + dd * 8])
                        out_bufs[obuf][
                            pl.ds(obase + dd * 128 + c * _L, _L)] = vals
                return 0
            lax.fori_loop(0, _JB * (_IW // 128), inner, 0)

        n_pre = 3
        in_flight = [in_copy(n, n % 4) for n in range(n_pre)]
        pending_out = [None, None, None]
        for n in range(_UNITS):
            ibuf, obuf = n % 4, n % 3
            in_flight.pop(0).wait()
            if n + n_pre < _UNITS:
                in_flight.append(in_copy(n + n_pre, (n + n_pre) % 4))
            if pending_out[obuf] is not None:
                for cp in pending_out[obuf]:
                    cp.wait()
            compute(ibuf, obuf)
            pending_out[obuf] = out_copies_for(n, obuf)

        for grp in pending_out:
            if grp is not None:
                for cp in grp:
                    cp.wait()

    return sc_gather


@functools.lru_cache(maxsize=None)
def _sc_gather_fn():
    return _make_sc_gather()


@jax.jit
def kernel(x, weight):
    wc = jnp.pad(weight, ((0, 8 - weight.shape[0]), (0, 0))).T.reshape(-1)
    f = _sc_gather_fn()(x.T, wc)
    return (f.reshape(_NJ, _NI // 128, _D, 128)
            .transpose(1, 3, 0, 2)
            .reshape(_NI, _NJ, _D))


# speedup vs baseline: 1.0687x; 1.0687x over previous
import functools

import jax
import jax.numpy as jnp
from jax import lax
from jax.experimental import pallas as pl
from jax.experimental.pallas import tpu as pltpu
from jax.experimental.pallas import tpu_sc as plsc

_NC = 2
_NS = 16
_NW = _NC * _NS
_L = 16

_NI = 16384
_NJ = 200
_D = 4
_JB = 8
_IW = _NI // _NW
_KI = _JB * _IW
_KO = _KI * _D
_OROW = _IW * _D
_UNITS = _NJ // _JB


def _make_sc_gather():
    mesh = plsc.VectorSubcoreMesh(
        core_axis_name="c", subcore_axis_name="s", num_cores=_NC,
        num_subcores=_NS)

    @functools.partial(
        pl.kernel,
        out_type=jax.ShapeDtypeStruct((_NI * _NJ * _D,), jnp.float32),
        mesh=mesh,
        compiler_params=pltpu.CompilerParams(needs_layout_passes=False),
        scratch_types=(
            [pltpu.VMEM((_JB, _IW), jnp.int32) for _ in range(4)]
            + [pltpu.VMEM((_KO,), jnp.float32) for _ in range(3)]
            + [
                pltpu.VMEM((32,), jnp.float32),
                pltpu.SemaphoreType.DMA,
                pltpu.SemaphoreType.DMA,
            ]
        ),
    )
    def sc_gather(xt_hbm, wc_hbm, out_hbm, idx0_v, idx1_v, idx2_v, idx3_v,
                  out0_v, out1_v, out2_v, w_v, in_sem, out_sem):
        idx_bufs = [idx0_v, idx1_v, idx2_v, idx3_v]
        out_bufs = [out0_v, out1_v, out2_v]
        wid = lax.axis_index("s") * _NC + lax.axis_index("c")
        i0 = wid * _IW
        pltpu.sync_copy(wc_hbm, w_v)

        def in_copy(n, buf):
            return pltpu.async_copy(
                xt_hbm.at[pl.ds(n * _JB, _JB), pl.ds(i0, _IW)],
                idx_bufs[buf], in_sem)

        def out_copies_for(n, buf):
            return [
                pltpu.async_copy(
                    out_bufs[buf].at[pl.ds(jj * _OROW, _OROW)],
                    out_hbm.at[pl.ds(
                        (n * _JB + jj) * (_NI * _D) + wid * _OROW, _OROW)],
                    out_sem)
                for jj in range(_JB)]

        tabs = [w_v.at[pl.ds(dd * 8, 8)] for dd in range(_D)]

        def compute(ibuf, obuf):
            def inner(m, _):
                jj = m // (_IW // 128)
                tt = m % (_IW // 128)
                ibase = tt * 128
                obase = jj * _OROW + tt * 512
                idxs = [
                    idx_bufs[ibuf][jj, pl.ds(ibase + c * _L, _L)]
                    for c in range(128 // _L)]
                for dd in range(_D):
                    for c in range(128 // _L):
                        out_bufs[obuf][
                            pl.ds(obase + dd * 128 + c * _L, _L)
                        ] = plsc.load_gather(tabs[dd], [idxs[c]])
                return 0
            lax.fori_loop(0, _JB * (_IW // 128), inner, 0)

        n_pre = 3
        in_flight = [in_copy(n, n % 4) for n in range(n_pre)]
        pending_out = [None, None, None]
        for n in range(_UNITS):
            ibuf, obuf = n % 4, n % 3
            in_flight.pop(0).wait()
            if n + n_pre < _UNITS:
                in_flight.append(in_copy(n + n_pre, (n + n_pre) % 4))
            if pending_out[obuf] is not None:
                for cp in pending_out[obuf]:
                    cp.wait()
            compute(ibuf, obuf)
            pending_out[obuf] = out_copies_for(n, obuf)

        for grp in pending_out:
            if grp is not None:
                for cp in grp:
                    cp.wait()

    return sc_gather


@functools.lru_cache(maxsize=None)
def _sc_gather_fn():
    return _make_sc_gather()


@jax.jit
def kernel(x, weight):
    wc = jnp.pad(weight, ((0, 8 - weight.shape[0]), (0, 0))).T.reshape(-1)
    f = _sc_gather_fn()(x.T, wc)
    return (f.reshape(_NJ, _NI // 128, _D, 128)
            .transpose(1, 3, 0, 2)
            .reshape(_NI, _NJ, _D))


# speedup vs baseline: 2.2758x; 2.1294x over previous
import functools

import jax
import jax.numpy as jnp
from jax import lax
from jax.experimental import pallas as pl
from jax.experimental.pallas import tpu as pltpu
from jax.experimental.pallas import tpu_sc as plsc

_NC = 2
_NS = 16
_NW = _NC * _NS
_L = 16

_NI = 16384
_NJ = 200
_D = 4
_JB = 8
_IW = _NI // _NW
_KI = _JB * _IW
_KO = _KI * _D
_OROW = _IW * _D
_UNITS = _NJ // _JB


def _make_sc_gather():
    mesh = plsc.VectorSubcoreMesh(
        core_axis_name="c", subcore_axis_name="s", num_cores=_NC,
        num_subcores=_NS)

    @functools.partial(
        pl.kernel,
        out_type=jax.ShapeDtypeStruct((_NI * _NJ * _D,), jnp.float32),
        mesh=mesh,
        compiler_params=pltpu.CompilerParams(needs_layout_passes=False),
        scratch_types=(
            [pltpu.VMEM((_JB, _IW), jnp.int32) for _ in range(4)]
            + [pltpu.VMEM((_KO,), jnp.float32) for _ in range(3)]
            + [
                pltpu.VMEM((32,), jnp.float32),
                pltpu.SemaphoreType.DMA,
                pltpu.SemaphoreType.DMA,
            ]
        ),
    )
    def sc_gather(xt_hbm, wc_hbm, out_hbm, idx0_v, idx1_v, idx2_v, idx3_v,
                  out0_v, out1_v, out2_v, w_v, in_sem, out_sem):
        idx_bufs = [idx0_v, idx1_v, idx2_v, idx3_v]
        out_bufs = [out0_v, out1_v, out2_v]
        wid = lax.axis_index("s") * _NC + lax.axis_index("c")
        i0 = wid * _IW
        pltpu.sync_copy(wc_hbm, w_v)

        def in_copy(n, buf):
            return pltpu.async_copy(
                xt_hbm.at[pl.ds(n * _JB, _JB), pl.ds(i0, _IW)],
                idx_bufs[buf], in_sem)

        def out_copies_for(n, buf):
            return [
                pltpu.async_copy(
                    out_bufs[buf].at[pl.ds(jj * _OROW, _OROW)],
                    out_hbm.at[pl.ds(
                        (n * _JB + jj) * (_NI * _D) + wid * _OROW, _OROW)],
                    out_sem)
                for jj in range(_JB)]

        tabs = [w_v.at[pl.ds(dd * 8, 8)] for dd in range(_D)]

        def compute(ibuf, obuf):
            def inner(m, _):
                jj = m // (_IW // 128)
                tt = m % (_IW // 128)
                ibase = tt * 128
                obase = jj * _OROW + tt * 512
                idxs = [
                    idx_bufs[ibuf][jj, pl.ds(ibase + c * _L, _L)]
                    for c in range(128 // _L)]
                for dd in range(_D):
                    vals = [
                        plsc.load_gather(tabs[dd], [idxs[c]])
                        for c in range(128 // _L)]
                    for c in range(128 // _L):
                        out_bufs[obuf][
                            pl.ds(obase + dd * 128 + c * _L, _L)] = vals[c]
                return 0
            lax.fori_loop(0, _JB * (_IW // 128), inner, 0)

        n_pre = 3
        in_flight = [in_copy(n, n % 4) for n in range(n_pre)]
        pending_out = [None, None, None]
        for n in range(_UNITS):
            ibuf, obuf = n % 4, n % 3
            in_flight.pop(0).wait()
            if n + n_pre < _UNITS:
                in_flight.append(in_copy(n + n_pre, (n + n_pre) % 4))
            if pending_out[obuf] is not None:
                for cp in pending_out[obuf]:
                    cp.wait()
            compute(ibuf, obuf)
            pending_out[obuf] = out_copies_for(n, obuf)

        for grp in pending_out:
            if grp is not None:
                for cp in grp:
                    cp.wait()

    return sc_gather


@functools.lru_cache(maxsize=None)
def _sc_gather_fn():
    return _make_sc_gather()


@jax.jit
def kernel(x, weight):
    wc = jnp.pad(weight, ((0, 8 - weight.shape[0]), (0, 0))).T.reshape(-1)
    f = _sc_gather_fn()(x.T, wc)
    return (f.reshape(_NJ, _NI // 128, _D, 128)
            .transpose(1, 3, 0, 2)
            .reshape(_NI, _NJ, _D))
